# baseline (speedup 1.0000x reference)
import functools

import jax
import jax.numpy as jnp
from jax import lax
from jax.experimental import pallas as pl
from jax.experimental.pallas import tpu as pltpu
from jax.experimental.pallas import tpu_sc as plsc

H = 512
W = 512
HW = H * W
DUMMY = HW

NC = 2
NS = 16
NWK = NC * NS
NB = 4
WPB = NWK // NB
PIX = HW // WPB

TCH = 2048
SCH = 4096
PCH = 256
GCH = 128
HUGE = 0x7FFFFFFF
LANES = 16


def _trunc(v):
    return jnp.where(v < 0, jnp.ceil(v), jnp.floor(v))


def _project_body(pts_ref, R_ref, T_ref, fcl_ref, prp_ref, pid_ref, z_ref,
                  *, n, chunk):
    ci = pl.program_id(1)
    pts = pts_ref[0]
    bf = lambda v: v.astype(jnp.bfloat16).astype(jnp.float32)
    X = bf(pts[0:1, :])
    Y = bf(pts[1:2, :])
    Z = bf(pts[2:3, :])
    r = lambda j, k: bf(R_ref[0, j, k])
    cam = lambda k: (X * r(0, k) + (Y * r(1, k) + Z * r(2, k)))
    cx = cam(0) + T_ref[0, 0, 0]
    cy = cam(1) + T_ref[0, 0, 1]
    cz = cam(2) + T_ref[0, 0, 2]
    x = fcl_ref[0, 0, 0] * cx / cz + prp_ref[0, 0, 0]
    y = fcl_ref[0, 0, 1] * cy / cz + prp_ref[0, 0, 1]
    s = (min(H, W) - 1) / 2.0
    ccx = -(W - 1) / 2.0
    ccy = -(H - 1) / 2.0
    sx = -(s * x + ccx)
    sy = -(s * y + ccy)
    fpx = _trunc(sx - 1e-06 + 0.5)
    fpy = _trunc(sy - 1e-06 + 0.5)
    inb = (fpx >= 0) & (fpx < W) & (fpy >= 0) & (fpy < H)
    vmask = cz > 0
    gidx = ci * chunk + lax.broadcasted_iota(jnp.int32, (1, chunk), 1)
    valid = inb & vmask & (gidx < n)
    fpx_c = jnp.clip(fpx, 0.0, W - 1.0)
    fpy_c = jnp.clip(fpy, 0.0, H - 1.0)
    pid = (fpy_c * W + fpx_c).astype(jnp.int32)
    pid_ref[...] = jnp.where(valid, pid, DUMMY)[None]
    z_ref[...] = cz[None]


def _raster_body(pid_hbm, z_hbm, feat_hbm, bg_hbm, out_hbm,
                 zmin, winner, pidb, zb, idxb0, idxb1, rows0, rows1,
                 outb0, outb1, bgv, gsem0, gsem1, osem0, osem1,
                 *, n, np_, d):
    cid = lax.axis_index("c")
    sid = lax.axis_index("s")
    wid = sid * NC + cid
    bi = wid // WPB
    base = (wid % WPB) * PIX
    nch = np_ // SCH
    dd = d + 1

    pltpu.sync_copy(bg_hbm, bgv)

    iota = lax.iota(jnp.int32, LANES)
    inf16 = jnp.full((LANES,), jnp.inf, jnp.float32)
    huge16 = jnp.full((LANES,), HUGE, jnp.int32)

    if True:
        def init_body(i, _):
            o = i * LANES
            zmin[pl.ds(o, LANES)] = inf16
            winner[pl.ds(o, LANES)] = huge16
            return 0
        lax.fori_loop(0, PIX // LANES, init_body, 0)

        def chAB(c, _):
            off = pl.multiple_of(c * SCH, SCH)
            pltpu.sync_copy(pid_hbm.at[bi, pl.ds(off, SCH)], pidb)
            pltpu.sync_copy(z_hbm.at[bi, pl.ds(off, SCH)], zb)
            row0 = bi * n + off

            def vec(i, _):
                grps = []
                for u in range(2):
                    o = (2 * i + u) * LANES
                    pv = pidb[pl.ds(o, LANES)]
                    zv = zb[pl.ds(o, LANES)]
                    li = pv - base
                    m = (li >= 0) & (li < PIX)
                    lis = jnp.where(m, li, 0)
                    iv = row0 + o + iota
                    cur = plsc.load_gather(zmin, [lis], mask=m)
                    wcur = plsc.load_gather(winner, [lis], mask=m)
                    want = m & ((zv < cur) | ((zv == cur) & (iv < wcur)))
                    grps.append((zv, lis, iv, want))
                for zv, lis, iv, want in grps:
                    def wbody(cand, zv=zv, lis=lis, iv=iv):
                        c2 = plsc.load_gather(zmin, [lis], mask=cand)
                        w2 = plsc.load_gather(winner, [lis], mask=cand)
                        w = cand & ((zv < c2) | ((zv == c2) & (iv < w2)))
                        plsc.store_scatter(zmin, [lis], zv, mask=w)
                        plsc.store_scatter(winner, [lis], iv, mask=w)
                        return w

                    lax.while_loop(jnp.any, wbody, want)
                return 0
            lax.fori_loop(0, SCH // (2 * LANES), vec, 0)
            return 0
        lax.fori_loop(0, nch, chAB, 0)

        idxbs = (idxb0, idxb1)
        rowss = (rows0, rows1)
        outbs = (outb0, outb1)
        gsems = (gsem0, gsem1)
        osems = (osem0, osem1)

        def build_idx(c, idxb):
            pbase = c * PCH

            def g_idx(g, _):
                o = g * LANES
                wv = winner[pl.ds(pbase + o, LANES)]
                has = wv != HUGE
                spread = base + pbase + o + iota
                idxb[pl.ds(o, LANES)] = jnp.where(has, wv, spread)
                return 0
            lax.fori_loop(0, PCH // LANES, g_idx, 0)

        def fire_gathers(idxb, rows, gsem):
            return [
                pltpu.async_copy(
                    feat_hbm.at[idxb.at[pl.ds(j * GCH, GCH)]],
                    rows.at[pl.ds(j * GCH, GCH)], gsem)
                for j in range(PCH // GCH)
            ]

        def assemble(c, rows, outb):
            pbase = c * PCH

            def g_emit(g, _):
                o = g * LANES
                wv = winner[pl.ds(pbase + o, LANES)]
                has = wv != HUGE
                rloc = o + iota
                for cc in range(d):
                    col = jnp.full((LANES,), cc, jnp.int32)
                    v = plsc.load_gather(rows, [rloc, col], mask=has)
                    bgcc = bgv[cc, pl.ds(0, LANES)]
                    outb[cc, pl.ds(o, LANES)] = jnp.where(has, v, bgcc)
                ones = jnp.where(has, 1.0, 0.0).astype(jnp.float32)
                outb[d, pl.ds(o, LANES)] = ones
                return 0
            lax.fori_loop(0, PCH // LANES, g_emit, 0)

        def out_dma(c, outb, osem):
            return pltpu.async_copy(
                outb, out_hbm.at[bi, :, pl.ds(base + c * PCH, PCH)], osem)

        def chC2(s, _):
            cs = (2 * s, 2 * s + 1)
            descs = []
            for p in (0, 1):
                build_idx(cs[p], idxbs[p])
                descs.append(fire_gathers(idxbs[p], rowss[p], gsems[p]))
            for p in (0, 1):
                @pl.when(s > 0)
                def _():
                    pltpu.make_async_copy(
                        outbs[p],
                        out_hbm.at[bi, :, pl.ds(base, PCH)],
                        osems[p]).wait()
                for cp in descs[p]:
                    cp.wait()
                assemble(cs[p], rowss[p], outbs[p])
                out_dma(cs[p], outbs[p], osems[p])
            return 0
        lax.fori_loop(0, PIX // PCH // 2, chC2, 0)
        for p in (0, 1):
            pltpu.make_async_copy(
                outbs[p], out_hbm.at[bi, :, pl.ds(base, PCH)], osems[p]).wait()


def kernel(points, features, R_row, T, fcl_ndc, prp_ndc, image_height,
           image_width, bg_feature):
    b, n, _ = points.shape
    d = features.shape[-1]
    chunk = TCH
    np_ = -(-n // SCH) * SCH
    nch = np_ // chunk

    pts_t = jnp.transpose(points, (0, 2, 1))
    pts_t = jnp.pad(pts_t, ((0, 0), (0, 0), (0, np_ - n)))

    proj = pl.pallas_call(
        functools.partial(_project_body, n=n, chunk=chunk),
        grid=(b, nch),
        in_specs=[
            pl.BlockSpec((1, 3, chunk), lambda bi, ci: (bi, 0, ci)),
            pl.BlockSpec((1, 3, 3), lambda bi, ci: (bi, 0, 0)),
            pl.BlockSpec((1, 1, 3), lambda bi, ci: (bi, 0, 0)),
            pl.BlockSpec((1, 1, 2), lambda bi, ci: (bi, 0, 0)),
            pl.BlockSpec((1, 1, 2), lambda bi, ci: (bi, 0, 0)),
        ],
        out_specs=[
            pl.BlockSpec((1, 1, chunk), lambda bi, ci: (bi * nch + ci, 0, 0)),
            pl.BlockSpec((1, 1, chunk), lambda bi, ci: (bi * nch + ci, 0, 0)),
        ],
        out_shape=[
            jax.ShapeDtypeStruct((b * nch, 1, chunk), jnp.int32),
            jax.ShapeDtypeStruct((b * nch, 1, chunk), jnp.float32),
        ],
    )
    pid, z = proj(pts_t, R_row, T[:, None, :], fcl_ndc[:, None, :],
                  prp_ndc[:, None, :])
    pid = pid.reshape(b, np_)
    z = z.reshape(b, np_)

    pid_off = ((jnp.asarray(image_height, jnp.int32) - H)
               + (jnp.asarray(image_width, jnp.int32) - W))
    pid = jnp.where(pid == DUMMY, DUMMY, pid + pid_off)

    feat2d = features.reshape(b * n, d)
    bg = jnp.broadcast_to(bg_feature.reshape(d)[:, None], (d, LANES))

    mesh = plsc.VectorSubcoreMesh(
        core_axis_name="c", subcore_axis_name="s",
        num_cores=NC, num_subcores=NS)
    raster = pl.kernel(
        functools.partial(_raster_body, n=n, np_=np_, d=d),
        out_type=jax.ShapeDtypeStruct((b, d + 1, HW), jnp.float32),
        mesh=mesh,
        compiler_params=pltpu.CompilerParams(
            needs_layout_passes=False, use_tc_tiling_on_sc=False),
        scratch_types=[
            pltpu.VMEM((PIX,), jnp.float32),
            pltpu.VMEM((PIX,), jnp.int32),
            pltpu.VMEM((SCH,), jnp.int32),
            pltpu.VMEM((SCH,), jnp.float32),
            pltpu.VMEM((PCH,), jnp.int32),
            pltpu.VMEM((PCH,), jnp.int32),
            pltpu.VMEM((PCH, d), jnp.float32),
            pltpu.VMEM((PCH, d), jnp.float32),
            pltpu.VMEM((d + 1, PCH), jnp.float32),
            pltpu.VMEM((d + 1, PCH), jnp.float32),
            pltpu.VMEM((d, LANES), jnp.float32),
            pltpu.SemaphoreType.DMA,
            pltpu.SemaphoreType.DMA,
            pltpu.SemaphoreType.DMA,
            pltpu.SemaphoreType.DMA,
        ],
    )
    out = raster(pid, z, feat2d, bg)
    return out.reshape(b, d + 1, H, W)


# speedup vs baseline: 1.0475x; 1.0475x over previous
import functools

import jax
import jax.numpy as jnp
from jax import lax
from jax.experimental import pallas as pl
from jax.experimental.pallas import tpu as pltpu
from jax.experimental.pallas import tpu_sc as plsc

H = 512
W = 512
HW = H * W
DUMMY = HW

NC = 2
NS = 16
NWK = NC * NS
NB = 4
WPB = NWK // NB
PIX = HW // WPB

TCH = 2048
SCH = 4096
PCH = 256
GCH = 128
HUGE = 0x7FFFFFFF
LANES = 16


def _trunc(v):
    return jnp.where(v < 0, jnp.ceil(v), jnp.floor(v))


def _project_body(pts_ref, R_ref, T_ref, fcl_ref, prp_ref, pid_ref, z_ref,
                  *, n, chunk):
    ci = pl.program_id(1)
    pts = pts_ref[0]
    bf = lambda v: v.astype(jnp.bfloat16).astype(jnp.float32)
    X = bf(pts[0:1, :])
    Y = bf(pts[1:2, :])
    Z = bf(pts[2:3, :])
    r = lambda j, k: bf(R_ref[0, j, k])
    cam = lambda k: (X * r(0, k) + (Y * r(1, k) + Z * r(2, k)))
    cx = cam(0) + T_ref[0, 0, 0]
    cy = cam(1) + T_ref[0, 0, 1]
    cz = cam(2) + T_ref[0, 0, 2]
    x = fcl_ref[0, 0, 0] * cx / cz + prp_ref[0, 0, 0]
    y = fcl_ref[0, 0, 1] * cy / cz + prp_ref[0, 0, 1]
    s = (min(H, W) - 1) / 2.0
    ccx = -(W - 1) / 2.0
    ccy = -(H - 1) / 2.0
    sx = -(s * x + ccx)
    sy = -(s * y + ccy)
    fpx = _trunc(sx - 1e-06 + 0.5)
    fpy = _trunc(sy - 1e-06 + 0.5)
    inb = (fpx >= 0) & (fpx < W) & (fpy >= 0) & (fpy < H)
    vmask = cz > 0
    gidx = ci * chunk + lax.broadcasted_iota(jnp.int32, (1, chunk), 1)
    valid = inb & vmask & (gidx < n)
    fpx_c = jnp.clip(fpx, 0.0, W - 1.0)
    fpy_c = jnp.clip(fpy, 0.0, H - 1.0)
    pid = (fpy_c * W + fpx_c).astype(jnp.int32)
    pid_ref[...] = jnp.where(valid, pid, DUMMY)[None]
    z_ref[...] = cz[None]


def _raster_body(pid_hbm, z_hbm, feat_hbm, bg_hbm, out_hbm,
                 zmin, winner, pidb, zb, idxb0, idxb1, rows0, rows1,
                 outb0, outb1, bgv, gsem0, gsem1, osem0, osem1,
                 *, n, np_, d):
    cid = lax.axis_index("c")
    sid = lax.axis_index("s")
    wid = sid * NC + cid
    bi = wid // WPB
    base = (wid % WPB) * PIX
    nch = np_ // SCH
    dd = d + 1

    pltpu.sync_copy(bg_hbm, bgv)

    iota = lax.iota(jnp.int32, LANES)
    inf16 = jnp.full((LANES,), jnp.inf, jnp.float32)
    huge16 = jnp.full((LANES,), HUGE, jnp.int32)

    if True:
        def init_body(i, _):
            o = i * LANES
            zmin[pl.ds(o, LANES)] = inf16
            winner[pl.ds(o, LANES)] = huge16
            return 0
        lax.fori_loop(0, PIX // LANES, init_body, 0)

        def chAB(c, _):
            off = pl.multiple_of(c * SCH, SCH)
            pltpu.sync_copy(pid_hbm.at[bi, pl.ds(off, SCH)], pidb)
            pltpu.sync_copy(z_hbm.at[bi, pl.ds(off, SCH)], zb)
            row0 = bi * n + off

            def vec(i, _):
                pv = pidb[pl.ds(i * LANES, LANES)]
                zv = zb[pl.ds(i * LANES, LANES)]
                li = pv - base
                m = (li >= 0) & (li < PIX)
                lis = jnp.where(m, li, 0)
                iv = row0 + i * LANES + iota
                cur = plsc.load_gather(zmin, [lis], mask=m)
                wcur = plsc.load_gather(winner, [lis], mask=m)
                want = m & ((zv < cur) | ((zv == cur) & (iv < wcur)))

                def wbody(wnt):
                    plsc.store_scatter(zmin, [lis], zv, mask=wnt)
                    plsc.store_scatter(winner, [lis], iv, mask=wnt)
                    c2 = plsc.load_gather(zmin, [lis], mask=wnt)
                    w2 = plsc.load_gather(winner, [lis], mask=wnt)
                    return wnt & ((zv < c2) | ((zv == c2) & (iv < w2)))

                lax.while_loop(jnp.any, wbody, want)
                return 0
            lax.fori_loop(0, SCH // LANES, vec, 0)
            return 0
        lax.fori_loop(0, nch, chAB, 0)

        idxbs = (idxb0, idxb1)
        rowss = (rows0, rows1)
        outbs = (outb0, outb1)
        gsems = (gsem0, gsem1)
        osems = (osem0, osem1)

        def build_idx(c, idxb):
            pbase = c * PCH

            def g_idx(g, _):
                o = g * LANES
                wv = winner[pl.ds(pbase + o, LANES)]
                has = wv != HUGE
                spread = base + pbase + o + iota
                idxb[pl.ds(o, LANES)] = jnp.where(has, wv, spread)
                return 0
            lax.fori_loop(0, PCH // LANES, g_idx, 0)

        def fire_gathers(idxb, rows, gsem):
            return [
                pltpu.async_copy(
                    feat_hbm.at[idxb.at[pl.ds(j * GCH, GCH)]],
                    rows.at[pl.ds(j * GCH, GCH)], gsem)
                for j in range(PCH // GCH)
            ]

        def assemble(c, rows, outb):
            pbase = c * PCH

            def g_emit(g, _):
                o = g * LANES
                wv = winner[pl.ds(pbase + o, LANES)]
                has = wv != HUGE
                rloc = o + iota
                for cc in range(d):
                    col = jnp.full((LANES,), cc, jnp.int32)
                    v = plsc.load_gather(rows, [rloc, col], mask=has)
                    bgcc = bgv[cc, pl.ds(0, LANES)]
                    outb[cc, pl.ds(o, LANES)] = jnp.where(has, v, bgcc)
                ones = jnp.where(has, 1.0, 0.0).astype(jnp.float32)
                outb[d, pl.ds(o, LANES)] = ones
                return 0
            lax.fori_loop(0, PCH // LANES, g_emit, 0)

        def out_dma(c, outb, osem):
            return pltpu.async_copy(
                outb, out_hbm.at[bi, :, pl.ds(base + c * PCH, PCH)], osem)

        def chC2(s, _):
            cs = (2 * s, 2 * s + 1)
            descs = []
            for p in (0, 1):
                build_idx(cs[p], idxbs[p])
                descs.append(fire_gathers(idxbs[p], rowss[p], gsems[p]))
            for p in (0, 1):
                @pl.when(s > 0)
                def _():
                    pltpu.make_async_copy(
                        outbs[p],
                        out_hbm.at[bi, :, pl.ds(base, PCH)],
                        osems[p]).wait()
                for cp in descs[p]:
                    cp.wait()
                assemble(cs[p], rowss[p], outbs[p])
                out_dma(cs[p], outbs[p], osems[p])
            return 0
        lax.fori_loop(0, PIX // PCH // 2, chC2, 0)
        for p in (0, 1):
            pltpu.make_async_copy(
                outbs[p], out_hbm.at[bi, :, pl.ds(base, PCH)], osems[p]).wait()


def kernel(points, features, R_row, T, fcl_ndc, prp_ndc, image_height,
           image_width, bg_feature):
    b, n, _ = points.shape
    d = features.shape[-1]
    chunk = TCH
    np_ = -(-n // SCH) * SCH
    nch = np_ // chunk

    pts_t = jnp.transpose(points, (0, 2, 1))
    pts_t = jnp.pad(pts_t, ((0, 0), (0, 0), (0, np_ - n)))

    proj = pl.pallas_call(
        functools.partial(_project_body, n=n, chunk=chunk),
        grid=(b, nch),
        in_specs=[
            pl.BlockSpec((1, 3, chunk), lambda bi, ci: (bi, 0, ci)),
            pl.BlockSpec((1, 3, 3), lambda bi, ci: (bi, 0, 0)),
            pl.BlockSpec((1, 1, 3), lambda bi, ci: (bi, 0, 0)),
            pl.BlockSpec((1, 1, 2), lambda bi, ci: (bi, 0, 0)),
            pl.BlockSpec((1, 1, 2), lambda bi, ci: (bi, 0, 0)),
        ],
        out_specs=[
            pl.BlockSpec((1, 1, chunk), lambda bi, ci: (bi * nch + ci, 0, 0)),
            pl.BlockSpec((1, 1, chunk), lambda bi, ci: (bi * nch + ci, 0, 0)),
        ],
        out_shape=[
            jax.ShapeDtypeStruct((b * nch, 1, chunk), jnp.int32),
            jax.ShapeDtypeStruct((b * nch, 1, chunk), jnp.float32),
        ],
    )
    pid, z = proj(pts_t, R_row, T[:, None, :], fcl_ndc[:, None, :],
                  prp_ndc[:, None, :])
    pid = pid.reshape(b, np_)
    z = z.reshape(b, np_)

    pid_off = ((jnp.asarray(image_height, jnp.int32) - H)
               + (jnp.asarray(image_width, jnp.int32) - W))
    pid = jnp.where(pid == DUMMY, DUMMY, pid + pid_off)

    feat2d = features.reshape(b * n, d)
    bg = jnp.broadcast_to(bg_feature.reshape(d)[:, None], (d, LANES))

    mesh = plsc.VectorSubcoreMesh(
        core_axis_name="c", subcore_axis_name="s",
        num_cores=NC, num_subcores=NS)
    raster = pl.kernel(
        functools.partial(_raster_body, n=n, np_=np_, d=d),
        out_type=jax.ShapeDtypeStruct((b, d + 1, HW), jnp.float32),
        mesh=mesh,
        compiler_params=pltpu.CompilerParams(
            needs_layout_passes=False, use_tc_tiling_on_sc=False),
        scratch_types=[
            pltpu.VMEM((PIX,), jnp.float32),
            pltpu.VMEM((PIX,), jnp.int32),
            pltpu.VMEM((SCH,), jnp.int32),
            pltpu.VMEM((SCH,), jnp.float32),
            pltpu.VMEM((PCH,), jnp.int32),
            pltpu.VMEM((PCH,), jnp.int32),
            pltpu.VMEM((PCH, d), jnp.float32),
            pltpu.VMEM((PCH, d), jnp.float32),
            pltpu.VMEM((d + 1, PCH), jnp.float32),
            pltpu.VMEM((d + 1, PCH), jnp.float32),
            pltpu.VMEM((d, LANES), jnp.float32),
            pltpu.SemaphoreType.DMA,
            pltpu.SemaphoreType.DMA,
            pltpu.SemaphoreType.DMA,
            pltpu.SemaphoreType.DMA,
        ],
    )
    out = raster(pid, z, feat2d, bg)
    return out.reshape(b, d + 1, H, W)
